# baseline (speedup 1.0000x reference)
import functools

import jax
import jax.numpy as jnp
import numpy as np
from jax import lax
from jax.experimental import pallas as pl
from jax.experimental.pallas import tpu as pltpu
from jax.experimental.pallas import tpu_sc as plsc

B = 1024
S = 200
TOK = 64
EMB = 64
MAX_TIME = 30.0

NUM_CORES = 2
NUM_SUBCORES = 16
NW = NUM_CORES * NUM_SUBCORES
BPW = B // NW

OUT_S = 2 * S - 1


def _pos_encodings_np():
    pos = np.arange(OUT_S, dtype=np.float32)
    freqs = np.arange(0, EMB, 2, dtype=np.float32)
    inv = 1.0 / (MAX_TIME ** (freqs / EMB))
    pe = pos[:, None] * inv[None, :]
    return np.concatenate([np.sin(pe), np.cos(pe)], axis=-1).astype(np.float32)


_BB = 8


def _tc_body(ex_ref, w_ref, b_ref, pe_ref, out_ref):
    x = ex_ref[...].reshape(_BB * S, TOK)
    w = w_ref[...]
    h = jax.lax.dot_general(x, w, (((1,), (1,)), ((), ())),
                            preferred_element_type=jnp.float32)
    h = h.reshape(_BB, S, EMB) + b_ref[...][None, :, :] + pe_ref[...][None, :, :]
    out_ref[...] = h


def _tc_encode(examples, W, b2d, pe_even):
    return pl.pallas_call(
        _tc_body,
        grid=(B // _BB,),
        in_specs=[
            pl.BlockSpec((_BB, S, TOK), lambda i: (i, 0, 0)),
            pl.BlockSpec((EMB, TOK), lambda i: (0, 0)),
            pl.BlockSpec((1, EMB), lambda i: (0, 0)),
            pl.BlockSpec((S, EMB), lambda i: (0, 0)),
        ],
        out_specs=pl.BlockSpec((_BB, S, EMB), lambda i: (i, 0, 0)),
        out_shape=jax.ShapeDtypeStruct((B, S, EMB), jnp.float32),
    )(examples, W, b2d, pe_even)


def _sc_body(labels_hbm, hex_hbm, pe_hbm, embs_hbm, out_hbm,
             idx_a, idx_b, lab_v, hex_v, pe_v, inter_v, gsem, lsem):
    wid = lax.axis_index("s") * NUM_CORES + lax.axis_index("c")

    pltpu.sync_copy(pe_hbm, pe_v)

    def per_batch(i, carry):
        bidx = wid * BPW + i
        pltpu.sync_copy(labels_hbm.at[bidx, pl.ds(0, 128)], idx_a)
        pltpu.sync_copy(labels_hbm.at[bidx, pl.ds(128, S - 128)], idx_b)
        ga = pltpu.async_copy(embs_hbm.at[idx_a], lab_v.at[pl.ds(0, 128)], gsem)
        gb = pltpu.async_copy(embs_hbm.at[idx_b],
                              lab_v.at[pl.ds(128, S - 128)], gsem)
        hx = pltpu.async_copy(hex_hbm.at[bidx], hex_v, lsem)
        ga.wait()
        gb.wait()
        hx.wait()

        def per_s(s, c):
            for j in range(EMB // 16):
                sl = pl.ds(j * 16, 16)
                inter_v[2 * s, sl] = hex_v[s, sl]
                inter_v[2 * s + 1, sl] = lab_v[s, sl] + pe_v[s, sl]
            return c

        lax.fori_loop(0, S - 1, per_s, 0, unroll=2)
        for j in range(EMB // 16):
            sl = pl.ds(j * 16, 16)
            inter_v[OUT_S - 1, sl] = hex_v[S - 1, sl]

        pltpu.sync_copy(inter_v, out_hbm.at[bidx])
        return carry

    lax.fori_loop(0, BPW, per_batch, 0)


def _sc_assemble(labels, hex_, pe_odd, embs):
    fn = pl.kernel(
        _sc_body,
        out_type=jax.ShapeDtypeStruct((B, OUT_S, EMB), jnp.float32),
        mesh=plsc.VectorSubcoreMesh(core_axis_name="c", subcore_axis_name="s",
                                    num_cores=NUM_CORES,
                                    num_subcores=NUM_SUBCORES),
        scratch_types=[
            pltpu.VMEM((128,), jnp.int32),
            pltpu.VMEM((S - 128,), jnp.int32),
            pltpu.VMEM((S, EMB), jnp.float32),
            pltpu.VMEM((S, EMB), jnp.float32),
            pltpu.VMEM((S - 1, EMB), jnp.float32),
            pltpu.VMEM((OUT_S, EMB), jnp.float32),
            pltpu.SemaphoreType.DMA,
            pltpu.SemaphoreType.DMA,
        ],
        compiler_params=pltpu.CompilerParams(use_tc_tiling_on_sc=False),
    )
    return fn(labels, hex_, pe_odd, embs)


def kernel(examples, labels, W, b, embs):
    pe = _pos_encodings_np()
    pe_even = jnp.asarray(pe[0::2])
    pe_odd = jnp.asarray(pe[1::2])
    hex_ = _tc_encode(examples, W, b.reshape(1, EMB), pe_even)
    return _sc_assemble(labels, hex_, pe_odd, embs)


# speedup vs baseline: 1.0338x; 1.0338x over previous
import functools

import jax
import jax.numpy as jnp
import numpy as np
from jax import lax
from jax.experimental import pallas as pl
from jax.experimental.pallas import tpu as pltpu
from jax.experimental.pallas import tpu_sc as plsc

B = 1024
S = 200
TOK = 64
EMB = 64
MAX_TIME = 30.0

NUM_CORES = 2
NUM_SUBCORES = 16
NW = NUM_CORES * NUM_SUBCORES
BPW = B // NW

OUT_S = 2 * S - 1


def _pos_encodings_np():
    pos = np.arange(OUT_S, dtype=np.float32)
    freqs = np.arange(0, EMB, 2, dtype=np.float32)
    inv = 1.0 / (MAX_TIME ** (freqs / EMB))
    pe = pos[:, None] * inv[None, :]
    return np.concatenate([np.sin(pe), np.cos(pe)], axis=-1).astype(np.float32)


_BB = 8


def _tc_body(ex_ref, w_ref, b_ref, pe_ref, out_ref):
    x = ex_ref[...].reshape(_BB * S, TOK)
    w = w_ref[...]
    h = jax.lax.dot_general(x, w, (((1,), (1,)), ((), ())),
                            preferred_element_type=jnp.float32)
    h = h.reshape(_BB, S, EMB) + b_ref[...][None, :, :] + pe_ref[...][None, :, :]
    out_ref[...] = h


def _tc_encode(examples, W, b2d, pe_even):
    return pl.pallas_call(
        _tc_body,
        grid=(B // _BB,),
        in_specs=[
            pl.BlockSpec((_BB, S, TOK), lambda i: (i, 0, 0)),
            pl.BlockSpec((EMB, TOK), lambda i: (0, 0)),
            pl.BlockSpec((1, EMB), lambda i: (0, 0)),
            pl.BlockSpec((S, EMB), lambda i: (0, 0)),
        ],
        out_specs=pl.BlockSpec((_BB, S, EMB), lambda i: (i, 0, 0)),
        out_shape=jax.ShapeDtypeStruct((B, S, EMB), jnp.float32),
    )(examples, W, b2d, pe_even)


def _sc_body(labels_hbm, hex_hbm, pe_hbm, embs_hbm, out_hbm,
             idx_a, idx_b, lab_v, hex_v, pe_v, inter_v, gsem, lsem):
    wid = lax.axis_index("s") * NUM_CORES + lax.axis_index("c")

    pltpu.sync_copy(pe_hbm, pe_v)

    def per_batch(i, carry):
        bidx = wid * BPW + i
        pltpu.sync_copy(labels_hbm.at[bidx, pl.ds(0, 128)], idx_a)
        pltpu.sync_copy(labels_hbm.at[bidx, pl.ds(128, S - 128)], idx_b)
        ga = pltpu.async_copy(embs_hbm.at[idx_a], lab_v.at[pl.ds(0, 128)], gsem)
        gb = pltpu.async_copy(embs_hbm.at[idx_b],
                              lab_v.at[pl.ds(128, S - 128)], gsem)
        hx = pltpu.async_copy(hex_hbm.at[bidx], hex_v, lsem)
        ga.wait()
        gb.wait()
        hx.wait()

        def per_s(s, c):
            for j in range(EMB // 16):
                sl = pl.ds(j * 16, 16)
                inter_v[2 * s, sl] = hex_v[s, sl]
                inter_v[2 * s + 1, sl] = lab_v[s, sl] + pe_v[s, sl]
            return c

        lax.fori_loop(0, S - 1, per_s, 0, unroll=2)
        for j in range(EMB // 16):
            sl = pl.ds(j * 16, 16)
            inter_v[OUT_S - 1, sl] = hex_v[S - 1, sl]

        pltpu.sync_copy(inter_v, out_hbm.at[bidx])
        return carry

    lax.fori_loop(0, BPW, per_batch, 0)


def _sc_assemble(labels, hex_, pe_odd, embs):
    fn = pl.kernel(
        _sc_body,
        out_type=jax.ShapeDtypeStruct((B, OUT_S, EMB), jnp.float32),
        mesh=plsc.VectorSubcoreMesh(core_axis_name="c", subcore_axis_name="s",
                                    num_cores=NUM_CORES,
                                    num_subcores=NUM_SUBCORES),
        scratch_types=[
            pltpu.VMEM((128,), jnp.int32),
            pltpu.VMEM((S - 128,), jnp.int32),
            pltpu.VMEM((S, 128), jnp.float32),
            pltpu.VMEM((S, EMB), jnp.float32),
            pltpu.VMEM((S - 1, EMB), jnp.float32),
            pltpu.VMEM((OUT_S, EMB), jnp.float32),
            pltpu.SemaphoreType.DMA,
            pltpu.SemaphoreType.DMA,
        ],
        compiler_params=pltpu.CompilerParams(use_tc_tiling_on_sc=False),
    )
    return fn(labels, hex_, pe_odd, embs)


def kernel(examples, labels, W, b, embs):
    pe = _pos_encodings_np()
    pe_even = jnp.asarray(pe[0::2])
    pe_odd = jnp.asarray(pe[1::2])
    embs128 = jnp.pad(embs, ((0, 0), (0, 128 - EMB)))
    hex_ = _tc_encode(examples, W, b.reshape(1, EMB), pe_even)
    return _sc_assemble(labels, hex_, pe_odd, embs128)


# speedup vs baseline: 1.2788x; 1.2371x over previous
import functools

import jax
import jax.numpy as jnp
import numpy as np
from jax import lax
from jax.experimental import pallas as pl
from jax.experimental.pallas import tpu as pltpu
from jax.experimental.pallas import tpu_sc as plsc

B = 1024
S = 200
TOK = 64
EMB = 64
MAX_TIME = 30.0

NUM_CORES = 2
NUM_SUBCORES = 16
NW = NUM_CORES * NUM_SUBCORES
BPW = B // NW

OUT_S = 2 * S - 1
HS = S // 2


def _pos_encodings_np():
    pos = np.arange(OUT_S, dtype=np.float32)
    freqs = np.arange(0, EMB, 2, dtype=np.float32)
    inv = 1.0 / (MAX_TIME ** (freqs / EMB))
    pe = pos[:, None] * inv[None, :]
    return np.concatenate([np.sin(pe), np.cos(pe)], axis=-1).astype(np.float32)


def _sc_body(labels_hbm, pairs_hbm, out_hbm, idx_v, hidx_v, g_v, pk_v,
             gsem):
    wid = lax.axis_index("s") * NUM_CORES + lax.axis_index("c")

    def per_batch(i, carry):
        bidx = wid * BPW + i
        pltpu.sync_copy(labels_hbm.at[bidx], idx_v.at[pl.ds(0, S)])
        for c in range(13):
            sl = pl.ds(c * 16, 16)
            hidx_v[sl] = idx_v[sl] >> 1
        ga = pltpu.async_copy(pairs_hbm.at[hidx_v.at[pl.ds(0, 128)]],
                              g_v.at[pl.ds(0, 128)], gsem)
        gb = pltpu.async_copy(pairs_hbm.at[hidx_v.at[pl.ds(128, S - 128)]],
                              g_v.at[pl.ds(128, S - 128)], gsem)
        ga.wait()
        gb.wait()

        def per_t(t, c):
            chunk = idx_v[pl.ds((t // 8) * 16, 16)]
            for half in range(2):
                s = 2 * t + half
                lane = jnp.broadcast_to(s % 16, (16,)).astype(jnp.int32)
                lbl = lax.gather(
                    chunk, lane[:, None],
                    dimension_numbers=lax.GatherDimensionNumbers(
                        offset_dims=(), collapsed_slice_dims=(0,),
                        start_index_map=(0,)),
                    slice_sizes=(1,),
                    mode=lax.GatherScatterMode.PROMISE_IN_BOUNDS)
                qf = (lbl & 1).astype(jnp.float32)
                pf = 1.0 - qf
                for j in range(EMB // 16):
                    lo = g_v[s, pl.ds(j * 16, 16)]
                    hi = g_v[s, pl.ds(EMB + j * 16, 16)]
                    v = lo * pf + hi * qf
                    pk_v[pl.ds(t * 128 + half * EMB + j * 16, 16)] = v
            return c

        lax.fori_loop(0, HS, per_t, 0, unroll=2)
        pltpu.sync_copy(pk_v, out_hbm.at[bidx])
        return carry

    lax.fori_loop(0, BPW, per_batch, 0)


def _sc_gather(labels, pairs):
    fn = pl.kernel(
        _sc_body,
        out_type=jax.ShapeDtypeStruct((B, HS * 128), jnp.float32),
        mesh=plsc.VectorSubcoreMesh(core_axis_name="c", subcore_axis_name="s",
                                    num_cores=NUM_CORES,
                                    num_subcores=NUM_SUBCORES),
        scratch_types=[
            pltpu.VMEM((208,), jnp.int32),
            pltpu.VMEM((208,), jnp.int32),
            pltpu.VMEM((S, 128), jnp.float32),
            pltpu.VMEM((HS * 128,), jnp.float32),
            pltpu.SemaphoreType.DMA,
        ],
        compiler_params=pltpu.CompilerParams(use_tc_tiling_on_sc=False),
    )
    return fn(labels, pairs)


def _tc_body(ex_ref, lab_ref, w_ref, b_ref, pe_ref, out_ref):
    ii = lax.broadcasted_iota(jnp.int32, (EMB, EMB), 0)
    jj = lax.broadcasted_iota(jnp.int32, (EMB, EMB), 1)
    eye = (ii == jj).astype(jnp.float32)
    w = w_ref[...]
    bvec = b_ref[...]
    x = ex_ref[...]
    labs = lab_ref[...]

    for half in range(2):
        h = lax.dot_general(w, x[half], (((1,), (0,)), ((), ())),
                            preferred_element_type=jnp.float32)
        out_ref[2 * half] = h + bvec + pe_ref[2 * half]
        l_bm = labs[:, half * EMB:(half + 1) * EMB]
        lt = lax.dot_general(eye, l_bm, (((1,), (1,)), ((), ())),
                             preferred_element_type=jnp.float32)
        out_ref[2 * half + 1] = lt + pe_ref[2 * half + 1]


def _tc_assemble(examples_t, h_lab, W, b2d, pe_t):
    return pl.pallas_call(
        _tc_body,
        grid=(HS,),
        in_specs=[
            pl.BlockSpec((2, TOK, B), lambda t: (t, 0, 0)),
            pl.BlockSpec((B, 128), lambda t: (0, t)),
            pl.BlockSpec((EMB, TOK), lambda t: (0, 0)),
            pl.BlockSpec((EMB, 1), lambda t: (0, 0)),
            pl.BlockSpec((4, EMB, 1), lambda t: (t, 0, 0)),
        ],
        out_specs=pl.BlockSpec((4, EMB, B), lambda t: (t, 0, 0)),
        out_shape=jax.ShapeDtypeStruct((OUT_S, EMB, B), jnp.float32),
    )(examples_t, h_lab, W, b2d, pe_t)


def kernel(examples, labels, W, b, embs):
    pe = _pos_encodings_np()
    pe_t = np.zeros((4 * HS, EMB, 1), dtype=np.float32)
    pe_t[:OUT_S, :, 0] = pe
    pairs = jnp.reshape(embs, (embs.shape[0] // 2, 2 * EMB))
    h_lab = _sc_gather(labels, pairs)
    examples_t = jnp.transpose(examples, (1, 2, 0))
    out_t = _tc_assemble(examples_t, h_lab, W, b.reshape(EMB, 1),
                         jnp.asarray(pe_t))
    return jnp.transpose(out_t, (2, 0, 1))


# speedup vs baseline: 1.7329x; 1.3551x over previous
import functools

import jax
import jax.numpy as jnp
import numpy as np
from jax import lax
from jax.experimental import pallas as pl
from jax.experimental.pallas import tpu as pltpu
from jax.experimental.pallas import tpu_sc as plsc

B = 1024
S = 200
TOK = 64
EMB = 64
MAX_TIME = 30.0

NUM_CORES = 2
NUM_SUBCORES = 16
NW = NUM_CORES * NUM_SUBCORES
BPW = B // NW

OUT_S = 2 * S - 1
HS = S // 2
NHALF = 500000


def _pos_encodings_np():
    pos = np.arange(OUT_S, dtype=np.float32)
    freqs = np.arange(0, EMB, 2, dtype=np.float32)
    inv = 1.0 / (MAX_TIME ** (freqs / EMB))
    pe = pos[:, None] * inv[None, :]
    return np.concatenate([np.sin(pe), np.cos(pe)], axis=-1).astype(np.float32)


def _sc_body(labels_hbm, pairs_hbm, out_hbm, idx_v, hidx_v, g_v, pk_v,
             gsem):
    wid = lax.axis_index("s") * NUM_CORES + lax.axis_index("c")

    def per_batch(i, carry):
        bidx = wid * BPW + i
        pltpu.sync_copy(labels_hbm.at[bidx], idx_v.at[pl.ds(0, S)])
        for c in range(13):
            sl = pl.ds(c * 16, 16)
            lbl = idx_v[sl]
            ge = jnp.int32(1) + ((lbl - jnp.int32(NHALF)) >> 31)
            hidx_v[sl] = lbl - ge * jnp.int32(NHALF)
        ga = pltpu.async_copy(pairs_hbm.at[hidx_v.at[pl.ds(0, 128)]],
                              g_v.at[pl.ds(0, 128)], gsem)
        gb = pltpu.async_copy(pairs_hbm.at[hidx_v.at[pl.ds(128, S - 128)]],
                              g_v.at[pl.ds(128, S - 128)], gsem)
        ga.wait()
        gb.wait()

        def per_t(t, c):
            chunk = idx_v[pl.ds((t // 8) * 16, 16)]
            for half in range(2):
                s = 2 * t + half
                lane = jnp.broadcast_to(s % 16, (16,)).astype(jnp.int32)
                lbl = lax.gather(
                    chunk, lane[:, None],
                    dimension_numbers=lax.GatherDimensionNumbers(
                        offset_dims=(), collapsed_slice_dims=(0,),
                        start_index_map=(0,)),
                    slice_sizes=(1,),
                    mode=lax.GatherScatterMode.PROMISE_IN_BOUNDS)
                ge = jnp.int32(1) + ((lbl - jnp.int32(NHALF)) >> 31)
                qf = ge.astype(jnp.float32)
                pf = 1.0 - qf
                for j in range(EMB // 16):
                    lo = g_v[s, pl.ds(j * 16, 16)]
                    hi = g_v[s, pl.ds(EMB + j * 16, 16)]
                    v = lo * pf + hi * qf
                    pk_v[pl.ds(t * 128 + half * EMB + j * 16, 16)] = v
            return c

        lax.fori_loop(0, HS, per_t, 0, unroll=2)
        pltpu.sync_copy(pk_v, out_hbm.at[bidx])
        return carry

    lax.fori_loop(0, BPW, per_batch, 0)


def _sc_gather(labels, pairs):
    fn = pl.kernel(
        _sc_body,
        out_type=jax.ShapeDtypeStruct((B, HS * 128), jnp.float32),
        mesh=plsc.VectorSubcoreMesh(core_axis_name="c", subcore_axis_name="s",
                                    num_cores=NUM_CORES,
                                    num_subcores=NUM_SUBCORES),
        scratch_types=[
            pltpu.VMEM((208,), jnp.int32),
            pltpu.VMEM((208,), jnp.int32),
            pltpu.VMEM((S, 128), jnp.float32),
            pltpu.VMEM((HS * 128,), jnp.float32),
            pltpu.SemaphoreType.DMA,
        ],
        compiler_params=pltpu.CompilerParams(use_tc_tiling_on_sc=False),
    )
    return fn(labels, pairs)


_CP = 2048


def _pairs_body(lo_ref, hi_ref, out_ref):
    lo = jnp.transpose(lo_ref[...], (1, 0))
    hi = jnp.transpose(hi_ref[...], (1, 0))
    out_ref[...] = jnp.concatenate([lo, hi], axis=1)


def _tc_pairs(embs_t):
    n = embs_t.shape[1]
    half_blocks = (n // 2) // _CP
    return pl.pallas_call(
        _pairs_body,
        grid=(half_blocks,),
        in_specs=[
            pl.BlockSpec((EMB, _CP), lambda i: (0, i)),
            pl.BlockSpec((EMB, _CP), lambda i: (0, i + half_blocks)),
        ],
        out_specs=pl.BlockSpec((_CP, 2 * EMB), lambda i: (i, 0)),
        out_shape=jax.ShapeDtypeStruct((n // 2, 2 * EMB), jnp.float32),
    )(embs_t, embs_t)


def _tc_body(ex_ref, lab_ref, w_ref, b_ref, pe_ref, out_ref):
    ii = lax.broadcasted_iota(jnp.int32, (EMB, EMB), 0)
    jj = lax.broadcasted_iota(jnp.int32, (EMB, EMB), 1)
    eye = (ii == jj).astype(jnp.float32)
    w = w_ref[...]
    bvec = b_ref[...]
    x = ex_ref[...]
    labs = lab_ref[...]

    for half in range(2):
        h = lax.dot_general(w, x[half], (((1,), (0,)), ((), ())),
                            preferred_element_type=jnp.float32)
        out_ref[2 * half] = h + bvec + pe_ref[2 * half]
        l_bm = labs[:, half * EMB:(half + 1) * EMB]
        lt = lax.dot_general(eye, l_bm, (((1,), (1,)), ((), ())),
                             preferred_element_type=jnp.float32)
        out_ref[2 * half + 1] = lt + pe_ref[2 * half + 1]


def _tc_assemble(examples_t, h_lab, W, b2d, pe_t):
    return pl.pallas_call(
        _tc_body,
        grid=(HS,),
        in_specs=[
            pl.BlockSpec((2, TOK, B), lambda t: (t, 0, 0)),
            pl.BlockSpec((B, 128), lambda t: (0, t)),
            pl.BlockSpec((EMB, TOK), lambda t: (0, 0)),
            pl.BlockSpec((EMB, 1), lambda t: (0, 0)),
            pl.BlockSpec((4, EMB, 1), lambda t: (t, 0, 0)),
        ],
        out_specs=pl.BlockSpec((4, EMB, B), lambda t: (t, 0, 0)),
        out_shape=jax.ShapeDtypeStruct((OUT_S, EMB, B), jnp.float32),
    )(examples_t, h_lab, W, b2d, pe_t)


def kernel(examples, labels, W, b, embs):
    pe = _pos_encodings_np()
    pe_t = np.zeros((4 * HS, EMB, 1), dtype=np.float32)
    pe_t[:OUT_S, :, 0] = pe
    pairs = _tc_pairs(jnp.transpose(embs))
    h_lab = _sc_gather(labels, pairs)
    examples_t = jnp.transpose(examples, (1, 2, 0))
    out_t = _tc_assemble(examples_t, h_lab, W, b.reshape(EMB, 1),
                         jnp.asarray(pe_t))
    return jnp.transpose(out_t, (2, 0, 1))


# speedup vs baseline: 2.0392x; 1.1767x over previous
import functools

import jax
import jax.numpy as jnp
import numpy as np
from jax import lax
from jax.experimental import pallas as pl
from jax.experimental.pallas import tpu as pltpu
from jax.experimental.pallas import tpu_sc as plsc

B = 1024
S = 200
TOK = 64
EMB = 64
MAX_TIME = 30.0

NUM_CORES = 2
NUM_SUBCORES = 16
NW = NUM_CORES * NUM_SUBCORES
BPW = B // NW

OUT_S = 2 * S - 1
HS = S // 2
NHALF = 500000


def _pos_encodings_np():
    pos = np.arange(OUT_S, dtype=np.float32)
    freqs = np.arange(0, EMB, 2, dtype=np.float32)
    inv = 1.0 / (MAX_TIME ** (freqs / EMB))
    pe = pos[:, None] * inv[None, :]
    return np.concatenate([np.sin(pe), np.cos(pe)], axis=-1).astype(np.float32)


TMAJ = 112


def _sc_body(labels_hbm, pairs_hbm, out_hbm, lab_v, hidx0, hidx1, g0, g1,
             pk_v, oidx_v, gsem0, gsem1):
    wid = lax.axis_index("s") * NUM_CORES + lax.axis_index("c")
    lanes = lax.iota(jnp.int32, 16)

    pltpu.sync_copy(labels_hbm.at[pl.ds(wid * BPW, BPW)],
                    lab_v.at[:, pl.ds(0, S)])

    hidx = (hidx0, hidx1)
    gbuf = (g0, g1)
    gsem = (gsem0, gsem1)

    def transform_and_issue(k, bb):
        for c in range(13):
            sl = pl.ds(c * 16, 16)
            lbl = lab_v[k, sl]
            ge = jnp.int32(1) + ((lbl - jnp.int32(NHALF)) >> 31)
            hidx[bb][sl] = lbl - ge * jnp.int32(NHALF)
        ga = pltpu.async_copy(pairs_hbm.at[hidx[bb].at[pl.ds(0, 128)]],
                              gbuf[bb].at[pl.ds(0, 128)], gsem[bb])
        gb = pltpu.async_copy(pairs_hbm.at[hidx[bb].at[pl.ds(128, S - 128)]],
                              gbuf[bb].at[pl.ds(128, S - 128)], gsem[bb])
        return ga, gb

    def compact(k, bb):
        g_v = gbuf[bb]

        def per_t(t, c):
            chunk = lab_v[k, pl.ds((t // 8) * 16, 16)]
            for half in range(2):
                s = 2 * t + half
                lane = jnp.broadcast_to(s % 16, (16,)).astype(jnp.int32)
                lbl = lax.gather(
                    chunk, lane[:, None],
                    dimension_numbers=lax.GatherDimensionNumbers(
                        offset_dims=(), collapsed_slice_dims=(0,),
                        start_index_map=(0,)),
                    slice_sizes=(1,),
                    mode=lax.GatherScatterMode.PROMISE_IN_BOUNDS)
                qf = (jnp.int32(1)
                      + ((lbl - jnp.int32(NHALF)) >> 31)).astype(jnp.float32)
                pf = 1.0 - qf
                for j in range(EMB // 16):
                    lo = g_v[s, pl.ds(j * 16, 16)]
                    hi = g_v[s, pl.ds(EMB + j * 16, 16)]
                    pk_v[t, pl.ds(half * EMB + j * 16, 16)] = lo * pf + hi * qf
            return c

        lax.fori_loop(0, HS, per_t, 0, unroll=2)

    def wait_g(bb):
        pltpu.make_async_copy(pairs_hbm.at[hidx[bb].at[pl.ds(0, 128)]],
                              gbuf[bb].at[pl.ds(0, 128)], gsem[bb]).wait()
        pltpu.make_async_copy(pairs_hbm.at[hidx[bb].at[pl.ds(128, S - 128)]],
                              gbuf[bb].at[pl.ds(128, S - 128)],
                              gsem[bb]).wait()

    def emit(k, bb):
        compact(k, bb)
        bidx = wid * BPW + k
        for c in range(TMAJ // 16):
            oidx_v[pl.ds(c * 16, 16)] = bidx + B * (c * 16 + lanes)
        pltpu.sync_copy(pk_v, out_hbm.at[oidx_v])

    transform_and_issue(0, 0)

    def macro(m, carry):
        wait_g(0)
        transform_and_issue(2 * m + 1, 1)
        emit(2 * m, 0)
        wait_g(1)

        @pl.when(m < BPW // 2 - 1)
        def _():
            transform_and_issue(2 * m + 2, 0)

        emit(2 * m + 1, 1)
        return carry

    lax.fori_loop(0, BPW // 2, macro, 0)


def _sc_gather(labels, pairs):
    fn = pl.kernel(
        _sc_body,
        out_type=jax.ShapeDtypeStruct((TMAJ * B, 128), jnp.float32),
        mesh=plsc.VectorSubcoreMesh(core_axis_name="c", subcore_axis_name="s",
                                    num_cores=NUM_CORES,
                                    num_subcores=NUM_SUBCORES),
        scratch_types=[
            pltpu.VMEM((BPW, 208), jnp.int32),
            pltpu.VMEM((208,), jnp.int32),
            pltpu.VMEM((208,), jnp.int32),
            pltpu.VMEM((S, 128), jnp.float32),
            pltpu.VMEM((S, 128), jnp.float32),
            pltpu.VMEM((TMAJ, 128), jnp.float32),
            pltpu.VMEM((TMAJ,), jnp.int32),
            pltpu.SemaphoreType.DMA,
            pltpu.SemaphoreType.DMA,
        ],
        compiler_params=pltpu.CompilerParams(use_tc_tiling_on_sc=False),
    )
    return fn(labels, pairs)


_CP = 2048


def _pairs_body(lo_ref, hi_ref, out_ref):
    lo = jnp.transpose(lo_ref[...], (1, 0))
    hi = jnp.transpose(hi_ref[...], (1, 0))
    out_ref[...] = jnp.concatenate([lo, hi], axis=1)


def _tc_pairs(embs_t):
    n = embs_t.shape[1]
    half_blocks = (n // 2) // _CP
    return pl.pallas_call(
        _pairs_body,
        grid=(half_blocks,),
        in_specs=[
            pl.BlockSpec((EMB, _CP), lambda i: (0, i)),
            pl.BlockSpec((EMB, _CP), lambda i: (0, i + half_blocks)),
        ],
        out_specs=pl.BlockSpec((_CP, 2 * EMB), lambda i: (i, 0)),
        out_shape=jax.ShapeDtypeStruct((n // 2, 2 * EMB), jnp.float32),
    )(embs_t, embs_t)


def _tc_body(ex_ref, lab_ref, w_ref, b_ref, pe_ref, out_ref):
    ii = lax.broadcasted_iota(jnp.int32, (EMB, EMB), 0)
    jj = lax.broadcasted_iota(jnp.int32, (EMB, EMB), 1)
    eye = (ii == jj).astype(jnp.float32)
    w = w_ref[...]
    bvec = b_ref[...]
    x = ex_ref[...]
    labs = lab_ref[...][0]

    for half in range(2):
        h = lax.dot_general(w, x[half], (((1,), (0,)), ((), ())),
                            preferred_element_type=jnp.float32)
        out_ref[2 * half] = h + bvec + pe_ref[2 * half]
        l_bm = labs[:, half * EMB:(half + 1) * EMB]
        lt = lax.dot_general(eye, l_bm, (((1,), (1,)), ((), ())),
                             preferred_element_type=jnp.float32)
        out_ref[2 * half + 1] = lt + pe_ref[2 * half + 1]


def _tc_assemble(examples_t, h_lab, W, b2d, pe_t):
    return pl.pallas_call(
        _tc_body,
        grid=(HS,),
        in_specs=[
            pl.BlockSpec((2, TOK, B), lambda t: (t, 0, 0)),
            pl.BlockSpec((1, B, 128), lambda t: (t, 0, 0)),
            pl.BlockSpec((EMB, TOK), lambda t: (0, 0)),
            pl.BlockSpec((EMB, 1), lambda t: (0, 0)),
            pl.BlockSpec((4, EMB, 1), lambda t: (t, 0, 0)),
        ],
        out_specs=pl.BlockSpec((4, EMB, B), lambda t: (t, 0, 0)),
        out_shape=jax.ShapeDtypeStruct((OUT_S, EMB, B), jnp.float32),
    )(examples_t, h_lab, W, b2d, pe_t)


def kernel(examples, labels, W, b, embs):
    pe = _pos_encodings_np()
    pe_t = np.zeros((4 * HS, EMB, 1), dtype=np.float32)
    pe_t[:OUT_S, :, 0] = pe
    pairs = _tc_pairs(jnp.transpose(embs))
    h_lab = jnp.reshape(_sc_gather(labels, pairs), (TMAJ, B, 128))
    examples_t = jnp.transpose(examples, (1, 2, 0))
    out_t = _tc_assemble(examples_t, h_lab, W, b.reshape(EMB, 1),
                         jnp.asarray(pe_t))
    return jnp.transpose(out_t, (2, 0, 1))
